# baseline (speedup 1.0000x reference)
import functools

import jax
import jax.numpy as jnp
from jax import lax
from jax.experimental import pallas as pl
from jax.experimental.pallas import tpu as pltpu
from jax.experimental.pallas import tpu_sc as plsc

N = 10000
E = 160000
D_IN = 256
DH = 512
EPS = 1e-5
CW = 128
NCH = 4
NC = 2
NS = 16
R = 2000
ROWS_PT = 624
EXTRA_OFF = ROWS_PT * NS
EXTRA = N - EXTRA_OFF
K = 40
EPT = E // NS
STEPS = EPT // K
NBUF = 5
KD = 40
DW = 128
EPW = E // (NC * NS)

_INTERPRET = False


def _sc_mesh():
    return plsc.VectorSubcoreMesh(core_axis_name="c", subcore_axis_name="s")


def _deg_call(ones_c, zeros_n, ei_flat):
    @functools.partial(
        pl.kernel,
        out_type=jax.ShapeDtypeStruct((NC, N, DW), jnp.float32),
        mesh=_sc_mesh(),
        scratch_types=[
            pltpu.VMEM_SHARED((N, DW), jnp.float32),
            pltpu.VMEM((KD, DW), jnp.float32),
            [pltpu.VMEM((KD,), jnp.int32) for _ in range(NBUF)],
            [pltpu.SemaphoreType.DMA for _ in range(NBUF)],
            [pltpu.SemaphoreType.DMA for _ in range(NBUF)],
        ],
    )
    def deg_kernel(ones_hbm, zeros_hbm, ei_hbm, deg_hbm, dsh, ones_v, didx,
                   dsem, ssem):
        cid = lax.axis_index("c")
        sid = lax.axis_index("s")
        rbase = sid * ROWS_PT
        pltpu.sync_copy(ones_hbm, ones_v)
        pltpu.sync_copy(zeros_hbm.at[pl.ds(rbase, ROWS_PT)],
                        dsh.at[pl.ds(rbase, ROWS_PT)])

        @pl.when(sid == NS - 1)
        def _():
            pltpu.sync_copy(zeros_hbm.at[pl.ds(EXTRA_OFF, EXTRA)],
                            dsh.at[pl.ds(EXTRA_OFF, EXTRA)])

        plsc.subcore_barrier()
        dst_hbm = ei_hbm
        wbase = E + (cid * NS + sid) * EPW
        dsteps = EPW // KD

        def start_didx(step, b):
            return pltpu.async_copy(
                dst_hbm.at[pl.ds(wbase + step * KD, KD)], didx[b], dsem[b])

        def start_scatter(b):
            return pltpu.async_copy(ones_v, dsh.at[didx[b]], ssem[b], add=True)

        for b in range(NBUF):
            start_didx(b, b)

        def body(g, carry):
            base = g * NBUF
            for b in range(NBUF):
                step = base + b
                pltpu.make_async_copy(
                    dst_hbm.at[pl.ds(wbase + step * KD, KD)], didx[b],
                    dsem[b]).wait()
                start_scatter(b).wait()
                start_didx(step + NBUF, b)
            return carry

        lax.fori_loop(0, dsteps // NBUF - 1, body, 0)
        sdescs = []
        for b in range(NBUF):
            step = dsteps - NBUF + b
            pltpu.make_async_copy(
                dst_hbm.at[pl.ds(wbase + step * KD, KD)], didx[b],
                dsem[b]).wait()
            sdescs.append(start_scatter(b))
        for d in sdescs:
            d.wait()
        plsc.subcore_barrier()
        pltpu.sync_copy(dsh.at[pl.ds(rbase, ROWS_PT)],
                        deg_hbm.at[cid, pl.ds(rbase, ROWS_PT)])

        @pl.when(sid == NS - 1)
        def _():
            pltpu.sync_copy(dsh.at[pl.ds(EXTRA_OFF, EXTRA)],
                            deg_hbm.at[cid, pl.ds(EXTRA_OFF, EXTRA)])

    return deg_kernel(ones_c, zeros_n, ei_flat)


def _agg_call(hws, ei_flat):
    @functools.partial(
        pl.kernel,
        out_type=jax.ShapeDtypeStruct((NCH, N, CW), jnp.float32),
        mesh=_sc_mesh(),
        scratch_types=[
            pltpu.VMEM_SHARED((N, CW), jnp.float32),
            [pltpu.VMEM((K, CW), jnp.float32) for _ in range(NBUF)],
            pltpu.VMEM((EPT,), jnp.int32),
            [pltpu.VMEM((K,), jnp.int32) for _ in range(NBUF)],
            [pltpu.SemaphoreType.DMA for _ in range(NBUF)],
            [pltpu.SemaphoreType.DMA for _ in range(NBUF)],
            [pltpu.SemaphoreType.DMA for _ in range(NBUF)],
        ],
    )
    def agg_kernel(hws_hbm, ei_hbm, z_hbm, zsh, rows, srcv, dsti,
                   gsem, ssem, dsem):
        cid = lax.axis_index("c")
        sid = lax.axis_index("s")
        rbase = sid * ROWS_PT
        ebase = sid * EPT
        dbase = E + ebase
        dst_hbm = ei_hbm
        pltpu.sync_copy(ei_hbm.at[pl.ds(ebase, EPT)], srcv)
        for q in range(NCH // NC):
            ch = cid * (NCH // NC) + q
            hws_ch = hws_hbm.at[ch]
            pltpu.sync_copy(hws_ch.at[pl.ds(rbase, ROWS_PT)],
                            zsh.at[pl.ds(rbase, ROWS_PT)])

            @pl.when(sid == NS - 1)
            def _():
                pltpu.sync_copy(hws_ch.at[pl.ds(EXTRA_OFF, EXTRA)],
                                zsh.at[pl.ds(EXTRA_OFF, EXTRA)])

            plsc.subcore_barrier()

            def start_didx(step, b):
                return pltpu.async_copy(
                    dst_hbm.at[pl.ds(dbase + step * K, K)], dsti[b], dsem[b])

            def start_gather(step, b):
                return pltpu.async_copy(
                    hws_ch.at[srcv.at[pl.ds(step * K, K)]], rows[b], gsem[b])

            def start_scatter(b):
                return pltpu.async_copy(
                    rows[b], zsh.at[dsti[b]], ssem[b], add=True)

            for b in range(NBUF):
                start_didx(b, b)
                start_gather(b, b)

            def body(g, carry):
                base = g * NBUF
                for b in range(NBUF):
                    step = base + b
                    pltpu.make_async_copy(
                        dst_hbm.at[pl.ds(dbase + step * K, K)], dsti[b],
                        dsem[b]).wait()
                    pltpu.make_async_copy(
                        hws_ch.at[srcv.at[pl.ds(step * K, K)]], rows[b],
                        gsem[b]).wait()
                    start_scatter(b)
                for b in range(NBUF):
                    step = base + b
                    pltpu.make_async_copy(
                        rows[b], zsh.at[dsti[b]], ssem[b]).wait()
                    start_didx(step + NBUF, b)
                    start_gather(step + NBUF, b)
                return carry

            lax.fori_loop(0, STEPS // NBUF - 1, body, 0)
            base = STEPS - NBUF
            sdescs = []
            for b in range(NBUF):
                step = base + b
                pltpu.make_async_copy(
                    dst_hbm.at[pl.ds(dbase + step * K, K)], dsti[b],
                    dsem[b]).wait()
                pltpu.make_async_copy(
                    hws_ch.at[srcv.at[pl.ds(step * K, K)]], rows[b],
                    gsem[b]).wait()
                sdescs.append(start_scatter(b))
            for d in sdescs:
                d.wait()
            plsc.subcore_barrier()
            pltpu.sync_copy(zsh.at[pl.ds(rbase, ROWS_PT)],
                            z_hbm.at[ch, pl.ds(rbase, ROWS_PT)])

            @pl.when(sid == NS - 1)
            def _():
                pltpu.sync_copy(zsh.at[pl.ds(EXTRA_OFF, EXTRA)],
                                z_hbm.at[ch, pl.ds(EXTRA_OFF, EXTRA)])

            plsc.subcore_barrier()

    return agg_kernel(hws, ei_flat)


def _a1_call(x, W1, deg_p):
    def a1(x_ref, w_ref, deg_ref, hws_ref, dis_ref):
        deg = deg_ref[0, :, 0:1] + deg_ref[1, :, 0:1] + 1.0
        dis = lax.rsqrt(deg)
        hw = jnp.dot(x_ref[...], w_ref[...], preferred_element_type=jnp.float32)
        hws_ref[0] = hw * dis
        dis_ref[...] = dis

    return pl.pallas_call(
        a1,
        grid=(N // R, NCH),
        in_specs=[
            pl.BlockSpec((R, D_IN), lambda i, d: (i, 0)),
            pl.BlockSpec((D_IN, CW), lambda i, d: (0, d)),
            pl.BlockSpec((NC, R, DW), lambda i, d: (0, i, 0)),
        ],
        out_specs=[
            pl.BlockSpec((1, R, CW), lambda i, d: (d, i, 0)),
            pl.BlockSpec((R, 1), lambda i, d: (i, 0)),
        ],
        out_shape=[
            jax.ShapeDtypeStruct((NCH, N, CW), jnp.float32),
            jax.ShapeDtypeStruct((N, 1), jnp.float32),
        ],
        interpret=_INTERPRET,
    )(x, W1, deg_p)


def _stats_call(z, dis, b4):
    def c_k(z_ref, dis_ref, b_ref, s_ref):
        i = pl.program_id(0)
        t = z_ref[...] * dis_ref[...][None] + b_ref[...][:, None, :]

        @pl.when(i == 0)
        def _():
            s_ref[...] = jnp.zeros_like(s_ref)

        s_ref[0] = s_ref[0] + jnp.sum(t, axis=1)
        s_ref[1] = s_ref[1] + jnp.sum(t * t, axis=1)

    return pl.pallas_call(
        c_k,
        grid=(N // R,),
        in_specs=[
            pl.BlockSpec((NCH, R, CW), lambda i: (0, i, 0)),
            pl.BlockSpec((R, 1), lambda i: (i, 0)),
            pl.BlockSpec((NCH, CW), lambda i: (0, 0)),
        ],
        out_specs=pl.BlockSpec((2, NCH, CW), lambda i: (0, 0, 0)),
        out_shape=jax.ShapeDtypeStruct((2, NCH, CW), jnp.float32),
        interpret=_INTERPRET,
    )(z, dis, b4)


def _a_call(z, dis, b4, sums, g4, be4, W):
    def a_k(z_ref, dis_ref, b_ref, s_ref, g_ref, be_ref, w_ref, o_ref):
        mu = s_ref[0] / N
        var = s_ref[1] / N - mu * mu
        scale = g_ref[...] * lax.rsqrt(var + EPS)
        shift = be_ref[...] - mu * scale
        dis = dis_ref[...]
        t = z_ref[...] * dis[None] + b_ref[...][:, None, :]
        h = jnp.maximum(t * scale[:, None, :] + shift[:, None, :], 0.0)
        acc = jnp.dot(h[0], w_ref[0:CW, :], preferred_element_type=jnp.float32)
        for c in range(1, NCH):
            acc = acc + jnp.dot(h[c], w_ref[c * CW:(c + 1) * CW, :],
                                preferred_element_type=jnp.float32)
        o_ref[0] = acc * dis

    return pl.pallas_call(
        a_k,
        grid=(N // R, NCH),
        in_specs=[
            pl.BlockSpec((NCH, R, CW), lambda i, d: (0, i, 0)),
            pl.BlockSpec((R, 1), lambda i, d: (i, 0)),
            pl.BlockSpec((NCH, CW), lambda i, d: (0, 0)),
            pl.BlockSpec((2, NCH, CW), lambda i, d: (0, 0, 0)),
            pl.BlockSpec((NCH, CW), lambda i, d: (0, 0)),
            pl.BlockSpec((NCH, CW), lambda i, d: (0, 0)),
            pl.BlockSpec((DH, CW), lambda i, d: (0, d)),
        ],
        out_specs=pl.BlockSpec((1, R, CW), lambda i, d: (d, i, 0)),
        out_shape=jax.ShapeDtypeStruct((NCH, N, CW), jnp.float32),
        interpret=_INTERPRET,
    )(z, dis, b4, sums, g4, be4, W)


def _d_call(z3, dis, b4, sums, g4, be4, Wc, bc11):
    def d_k(z_ref, dis_ref, b_ref, s_ref, g_ref, be_ref, wc_ref, bc_ref,
            h_ref, o_ref):
        mu = s_ref[0] / N
        var = s_ref[1] / N - mu * mu
        scale = g_ref[...] * lax.rsqrt(var + EPS)
        shift = be_ref[...] - mu * scale
        dis = dis_ref[...]
        oacc = jnp.zeros((R, 1), jnp.float32)
        for c in range(NCH):
            t = z_ref[c] * dis + b_ref[c][None]
            h = jnp.maximum(t * scale[c][None] + shift[c][None], 0.0)
            h_ref[:, c:c + 1, :] = h[:, None, :]
            oacc = oacc + jnp.dot(h, wc_ref[c * CW:(c + 1) * CW, :],
                                  preferred_element_type=jnp.float32)
        o_ref[...] = oacc + bc_ref[...]

    return pl.pallas_call(
        d_k,
        grid=(N // R,),
        in_specs=[
            pl.BlockSpec((NCH, R, CW), lambda i: (0, i, 0)),
            pl.BlockSpec((R, 1), lambda i: (i, 0)),
            pl.BlockSpec((NCH, CW), lambda i: (0, 0)),
            pl.BlockSpec((2, NCH, CW), lambda i: (0, 0, 0)),
            pl.BlockSpec((NCH, CW), lambda i: (0, 0)),
            pl.BlockSpec((NCH, CW), lambda i: (0, 0)),
            pl.BlockSpec((DH, 1), lambda i: (0, 0)),
            pl.BlockSpec((1, 1), lambda i: (0, 0)),
        ],
        out_specs=[
            pl.BlockSpec((R, NCH, CW), lambda i: (i, 0, 0)),
            pl.BlockSpec((R, 1), lambda i: (i, 0)),
        ],
        out_shape=[
            jax.ShapeDtypeStruct((N, NCH, CW), jnp.float32),
            jax.ShapeDtypeStruct((N, 1), jnp.float32),
        ],
        interpret=_INTERPRET,
    )(z3, dis, b4, sums, g4, be4, Wc, bc11)


def kernel(x, edge_index, W1, b1, g1, be1, W2, b2, g2, be2, W3, b3, g3, be3,
           Wc, bc):
    ei_flat = edge_index.reshape(2 * E)
    ones_c = jnp.ones((KD, DW), jnp.float32)
    zeros_n = jnp.zeros((N, DW), jnp.float32)
    b1r, g1r, be1r = (v.reshape(NCH, CW) for v in (b1, g1, be1))
    b2r, g2r, be2r = (v.reshape(NCH, CW) for v in (b2, g2, be2))
    b3r, g3r, be3r = (v.reshape(NCH, CW) for v in (b3, g3, be3))

    deg_p = _deg_call(ones_c, zeros_n, ei_flat)
    hws1, dis = _a1_call(x, W1, deg_p)
    z1 = _agg_call(hws1, ei_flat)
    s1 = _stats_call(z1, dis, b1r)
    hws2 = _a_call(z1, dis, b1r, s1, g1r, be1r, W2)
    z2 = _agg_call(hws2, ei_flat)
    s2 = _stats_call(z2, dis, b2r)
    hws3 = _a_call(z2, dis, b2r, s2, g2r, be2r, W3)
    z3 = _agg_call(hws3, ei_flat)
    s3 = _stats_call(z3, dis, b3r)
    h3d, out = _d_call(z3, dis, b3r, s3, g3r, be3r, Wc, bc.reshape(1, 1))
    return (out.reshape(N), h3d.reshape(N, DH))


# speedup vs baseline: 1.1563x; 1.1563x over previous
import functools

import jax
import jax.numpy as jnp
from jax import lax
from jax.experimental import pallas as pl
from jax.experimental.pallas import tpu as pltpu
from jax.experimental.pallas import tpu_sc as plsc

N = 10000
E = 160000
D_IN = 256
DH = 512
EPS = 1e-5
CW = 128
NCH = 4
NC = 2
NS = 16
R = 2000
ROWS_PT = 624
EXTRA_OFF = ROWS_PT * NS
EXTRA = N - EXTRA_OFF
K = 40
EPT = E // NS
STEPS = EPT // K
NBUF = 5
KD = 40
DW = 128
EPW = E // (NC * NS)

_INTERPRET = False


def _sc_mesh():
    return plsc.VectorSubcoreMesh(core_axis_name="c", subcore_axis_name="s")


def _deg_call(ones_c, zeros_n, ei_flat):
    @functools.partial(
        pl.kernel,
        out_type=jax.ShapeDtypeStruct((NC, N, DW), jnp.float32),
        mesh=_sc_mesh(),
        scratch_types=[
            pltpu.VMEM_SHARED((N, DW), jnp.float32),
            pltpu.VMEM((KD, DW), jnp.float32),
            [pltpu.VMEM((KD,), jnp.int32) for _ in range(NBUF)],
            [pltpu.SemaphoreType.DMA for _ in range(NBUF)],
            [pltpu.SemaphoreType.DMA for _ in range(NBUF)],
        ],
    )
    def deg_kernel(ones_hbm, zeros_hbm, ei_hbm, deg_hbm, dsh, ones_v, didx,
                   dsem, ssem):
        cid = lax.axis_index("c")
        sid = lax.axis_index("s")
        rbase = sid * ROWS_PT
        pltpu.sync_copy(ones_hbm, ones_v)
        pltpu.sync_copy(zeros_hbm.at[pl.ds(rbase, ROWS_PT)],
                        dsh.at[pl.ds(rbase, ROWS_PT)])

        @pl.when(sid == NS - 1)
        def _():
            pltpu.sync_copy(zeros_hbm.at[pl.ds(EXTRA_OFF, EXTRA)],
                            dsh.at[pl.ds(EXTRA_OFF, EXTRA)])

        plsc.subcore_barrier()
        dst_hbm = ei_hbm
        wbase = E + (cid * NS + sid) * EPW
        dsteps = EPW // KD

        def start_didx(step, b):
            return pltpu.async_copy(
                dst_hbm.at[pl.ds(wbase + step * KD, KD)], didx[b], dsem[b])

        def start_scatter(b):
            return pltpu.async_copy(ones_v, dsh.at[didx[b]], ssem[b], add=True)

        for b in range(NBUF):
            start_didx(b, b)

        def body(g, carry):
            base = g * NBUF
            for b in range(NBUF):
                step = base + b
                pltpu.make_async_copy(
                    dst_hbm.at[pl.ds(wbase + step * KD, KD)], didx[b],
                    dsem[b]).wait()
                start_scatter(b).wait()
                start_didx(step + NBUF, b)
            return carry

        lax.fori_loop(0, dsteps // NBUF - 1, body, 0)
        sdescs = []
        for b in range(NBUF):
            step = dsteps - NBUF + b
            pltpu.make_async_copy(
                dst_hbm.at[pl.ds(wbase + step * KD, KD)], didx[b],
                dsem[b]).wait()
            sdescs.append(start_scatter(b))
        for d in sdescs:
            d.wait()
        plsc.subcore_barrier()
        pltpu.sync_copy(dsh.at[pl.ds(rbase, ROWS_PT)],
                        deg_hbm.at[cid, pl.ds(rbase, ROWS_PT)])

        @pl.when(sid == NS - 1)
        def _():
            pltpu.sync_copy(dsh.at[pl.ds(EXTRA_OFF, EXTRA)],
                            deg_hbm.at[cid, pl.ds(EXTRA_OFF, EXTRA)])

    return deg_kernel(ones_c, zeros_n, ei_flat)


def _agg_call(hws, ei_flat):
    @functools.partial(
        pl.kernel,
        out_type=jax.ShapeDtypeStruct((NCH, N, CW), jnp.float32),
        mesh=_sc_mesh(),
        scratch_types=[
            pltpu.VMEM_SHARED((N, CW), jnp.float32),
            [pltpu.VMEM((K, CW), jnp.float32) for _ in range(NBUF)],
            pltpu.VMEM((EPT,), jnp.int32),
            [pltpu.VMEM((K,), jnp.int32) for _ in range(NBUF)],
            [pltpu.SemaphoreType.DMA for _ in range(NBUF)],
            [pltpu.SemaphoreType.DMA for _ in range(NBUF)],
            [pltpu.SemaphoreType.DMA for _ in range(NBUF)],
        ],
    )
    def agg_kernel(hws_hbm, ei_hbm, z_hbm, zsh, rows, srcv, dsti,
                   gsem, ssem, dsem):
        cid = lax.axis_index("c")
        sid = lax.axis_index("s")
        rbase = sid * ROWS_PT
        ebase = sid * EPT
        dbase = E + ebase
        dst_hbm = ei_hbm
        pltpu.sync_copy(ei_hbm.at[pl.ds(ebase, EPT)], srcv)
        for q in range(NCH // NC):
            ch = cid * (NCH // NC) + q
            hws_ch = hws_hbm.at[ch]
            pltpu.sync_copy(hws_ch.at[pl.ds(rbase, ROWS_PT)],
                            zsh.at[pl.ds(rbase, ROWS_PT)])

            @pl.when(sid == NS - 1)
            def _():
                pltpu.sync_copy(hws_ch.at[pl.ds(EXTRA_OFF, EXTRA)],
                                zsh.at[pl.ds(EXTRA_OFF, EXTRA)])

            plsc.subcore_barrier()

            def start_didx(step, b):
                return pltpu.async_copy(
                    dst_hbm.at[pl.ds(dbase + step * K, K)], dsti[b], dsem[b])

            def start_gather(step, b):
                return pltpu.async_copy(
                    hws_ch.at[srcv.at[pl.ds(step * K, K)]], rows[b], gsem[b])

            def start_scatter(b):
                return pltpu.async_copy(
                    rows[b], zsh.at[dsti[b]], ssem[b], add=True)

            for b in range(NBUF):
                start_didx(b, b)
                start_gather(b, b)

            def body(g, carry):
                base = g * NBUF
                for b in range(NBUF):
                    step = base + b
                    pltpu.make_async_copy(
                        dst_hbm.at[pl.ds(dbase + step * K, K)], dsti[b],
                        dsem[b]).wait()
                    pltpu.make_async_copy(
                        hws_ch.at[srcv.at[pl.ds(step * K, K)]], rows[b],
                        gsem[b]).wait()
                    start_scatter(b).wait()
                    start_didx(step + NBUF, b)
                    start_gather(step + NBUF, b)
                return carry

            lax.fori_loop(0, STEPS // NBUF - 1, body, 0)
            base = STEPS - NBUF
            sdescs = []
            for b in range(NBUF):
                step = base + b
                pltpu.make_async_copy(
                    dst_hbm.at[pl.ds(dbase + step * K, K)], dsti[b],
                    dsem[b]).wait()
                pltpu.make_async_copy(
                    hws_ch.at[srcv.at[pl.ds(step * K, K)]], rows[b],
                    gsem[b]).wait()
                sdescs.append(start_scatter(b))
            for d in sdescs:
                d.wait()
            plsc.subcore_barrier()
            pltpu.sync_copy(zsh.at[pl.ds(rbase, ROWS_PT)],
                            z_hbm.at[ch, pl.ds(rbase, ROWS_PT)])

            @pl.when(sid == NS - 1)
            def _():
                pltpu.sync_copy(zsh.at[pl.ds(EXTRA_OFF, EXTRA)],
                                z_hbm.at[ch, pl.ds(EXTRA_OFF, EXTRA)])

            plsc.subcore_barrier()

    return agg_kernel(hws, ei_flat)


def _a1_call(x, W1, deg_p):
    def a1(x_ref, w_ref, deg_ref, hws_ref, dis_ref):
        deg = deg_ref[0, :, 0:1] + deg_ref[1, :, 0:1] + 1.0
        dis = lax.rsqrt(deg)
        hw = jnp.dot(x_ref[...], w_ref[...], preferred_element_type=jnp.float32)
        hws_ref[0] = hw * dis
        dis_ref[...] = dis

    return pl.pallas_call(
        a1,
        grid=(N // R, NCH),
        in_specs=[
            pl.BlockSpec((R, D_IN), lambda i, d: (i, 0)),
            pl.BlockSpec((D_IN, CW), lambda i, d: (0, d)),
            pl.BlockSpec((NC, R, DW), lambda i, d: (0, i, 0)),
        ],
        out_specs=[
            pl.BlockSpec((1, R, CW), lambda i, d: (d, i, 0)),
            pl.BlockSpec((R, 1), lambda i, d: (i, 0)),
        ],
        out_shape=[
            jax.ShapeDtypeStruct((NCH, N, CW), jnp.float32),
            jax.ShapeDtypeStruct((N, 1), jnp.float32),
        ],
        interpret=_INTERPRET,
    )(x, W1, deg_p)


def _stats_call(z, dis, b4):
    def c_k(z_ref, dis_ref, b_ref, s_ref):
        i = pl.program_id(0)
        t = z_ref[...] * dis_ref[...][None] + b_ref[...][:, None, :]

        @pl.when(i == 0)
        def _():
            s_ref[...] = jnp.zeros_like(s_ref)

        s_ref[0] = s_ref[0] + jnp.sum(t, axis=1)
        s_ref[1] = s_ref[1] + jnp.sum(t * t, axis=1)

    return pl.pallas_call(
        c_k,
        grid=(N // R,),
        in_specs=[
            pl.BlockSpec((NCH, R, CW), lambda i: (0, i, 0)),
            pl.BlockSpec((R, 1), lambda i: (i, 0)),
            pl.BlockSpec((NCH, CW), lambda i: (0, 0)),
        ],
        out_specs=pl.BlockSpec((2, NCH, CW), lambda i: (0, 0, 0)),
        out_shape=jax.ShapeDtypeStruct((2, NCH, CW), jnp.float32),
        interpret=_INTERPRET,
    )(z, dis, b4)


def _a_call(z, dis, b4, sums, g4, be4, W):
    def a_k(z_ref, dis_ref, b_ref, s_ref, g_ref, be_ref, w_ref, o_ref):
        mu = s_ref[0] / N
        var = s_ref[1] / N - mu * mu
        scale = g_ref[...] * lax.rsqrt(var + EPS)
        shift = be_ref[...] - mu * scale
        dis = dis_ref[...]
        t = z_ref[...] * dis[None] + b_ref[...][:, None, :]
        h = jnp.maximum(t * scale[:, None, :] + shift[:, None, :], 0.0)
        acc = jnp.dot(h[0], w_ref[0:CW, :], preferred_element_type=jnp.float32)
        for c in range(1, NCH):
            acc = acc + jnp.dot(h[c], w_ref[c * CW:(c + 1) * CW, :],
                                preferred_element_type=jnp.float32)
        o_ref[0] = acc * dis

    return pl.pallas_call(
        a_k,
        grid=(N // R, NCH),
        in_specs=[
            pl.BlockSpec((NCH, R, CW), lambda i, d: (0, i, 0)),
            pl.BlockSpec((R, 1), lambda i, d: (i, 0)),
            pl.BlockSpec((NCH, CW), lambda i, d: (0, 0)),
            pl.BlockSpec((2, NCH, CW), lambda i, d: (0, 0, 0)),
            pl.BlockSpec((NCH, CW), lambda i, d: (0, 0)),
            pl.BlockSpec((NCH, CW), lambda i, d: (0, 0)),
            pl.BlockSpec((DH, CW), lambda i, d: (0, d)),
        ],
        out_specs=pl.BlockSpec((1, R, CW), lambda i, d: (d, i, 0)),
        out_shape=jax.ShapeDtypeStruct((NCH, N, CW), jnp.float32),
        interpret=_INTERPRET,
    )(z, dis, b4, sums, g4, be4, W)


def _d_call(z3, dis, b4, sums, g4, be4, Wc, bc11):
    def d_k(z_ref, dis_ref, b_ref, s_ref, g_ref, be_ref, wc_ref, bc_ref,
            h_ref, o_ref):
        mu = s_ref[0] / N
        var = s_ref[1] / N - mu * mu
        scale = g_ref[...] * lax.rsqrt(var + EPS)
        shift = be_ref[...] - mu * scale
        dis = dis_ref[...]
        oacc = jnp.zeros((R, 1), jnp.float32)
        for c in range(NCH):
            t = z_ref[c] * dis + b_ref[c][None]
            h = jnp.maximum(t * scale[c][None] + shift[c][None], 0.0)
            h_ref[:, c:c + 1, :] = h[:, None, :]
            oacc = oacc + jnp.dot(h, wc_ref[c * CW:(c + 1) * CW, :],
                                  preferred_element_type=jnp.float32)
        o_ref[...] = oacc + bc_ref[...]

    return pl.pallas_call(
        d_k,
        grid=(N // R,),
        in_specs=[
            pl.BlockSpec((NCH, R, CW), lambda i: (0, i, 0)),
            pl.BlockSpec((R, 1), lambda i: (i, 0)),
            pl.BlockSpec((NCH, CW), lambda i: (0, 0)),
            pl.BlockSpec((2, NCH, CW), lambda i: (0, 0, 0)),
            pl.BlockSpec((NCH, CW), lambda i: (0, 0)),
            pl.BlockSpec((NCH, CW), lambda i: (0, 0)),
            pl.BlockSpec((DH, 1), lambda i: (0, 0)),
            pl.BlockSpec((1, 1), lambda i: (0, 0)),
        ],
        out_specs=[
            pl.BlockSpec((R, NCH, CW), lambda i: (i, 0, 0)),
            pl.BlockSpec((R, 1), lambda i: (i, 0)),
        ],
        out_shape=[
            jax.ShapeDtypeStruct((N, NCH, CW), jnp.float32),
            jax.ShapeDtypeStruct((N, 1), jnp.float32),
        ],
        interpret=_INTERPRET,
    )(z3, dis, b4, sums, g4, be4, Wc, bc11)


def kernel(x, edge_index, W1, b1, g1, be1, W2, b2, g2, be2, W3, b3, g3, be3,
           Wc, bc):
    ei_flat = edge_index.reshape(2 * E)
    ones_c = jnp.ones((KD, DW), jnp.float32)
    zeros_n = jnp.zeros((N, DW), jnp.float32)
    b1r, g1r, be1r = (v.reshape(NCH, CW) for v in (b1, g1, be1))
    b2r, g2r, be2r = (v.reshape(NCH, CW) for v in (b2, g2, be2))
    b3r, g3r, be3r = (v.reshape(NCH, CW) for v in (b3, g3, be3))

    deg_p = _deg_call(ones_c, zeros_n, ei_flat)
    hws1, dis = _a1_call(x, W1, deg_p)
    z1 = _agg_call(hws1, ei_flat)
    s1 = _stats_call(z1, dis, b1r)
    hws2 = _a_call(z1, dis, b1r, s1, g1r, be1r, W2)
    z2 = _agg_call(hws2, ei_flat)
    s2 = _stats_call(z2, dis, b2r)
    hws3 = _a_call(z2, dis, b2r, s2, g2r, be2r, W3)
    z3 = _agg_call(hws3, ei_flat)
    s3 = _stats_call(z3, dis, b3r)
    h3d, out = _d_call(z3, dis, b3r, s3, g3r, be3r, Wc, bc.reshape(1, 1))
    return (out.reshape(N), h3d.reshape(N, DH))
